# baseline (speedup 1.0000x reference)
import functools

import jax
import jax.numpy as jnp
from jax import lax
from jax.experimental import pallas as pl
from jax.experimental.pallas import tpu as pltpu
from jax.experimental.pallas import tpu_sc as plsc

S = 256
K = 8
C = 16
LANES = 16
NC, NS = 2, 16
NW = NC * NS
ROWS_W = S // NW
PIX_W = ROWS_W * S
CH = 2048
R_NDC = 1.5 / S * 2.0
R2 = R_NDC * R_NDC


def _scalar(v):
    v = jnp.asarray(v)
    return v[0] if v.ndim == 1 else v


@functools.lru_cache(maxsize=None)
def _make_raster(B, N):
    mesh = plsc.VectorSubcoreMesh(
        core_axis_name="c", subcore_axis_name="s",
        num_cores=NC, num_subcores=NS)

    chunk_buf = [
        pltpu.VMEM((CH,), jnp.float32),
        pltpu.VMEM((CH,), jnp.float32),
        pltpu.VMEM((CH,), jnp.float32),
        pltpu.VMEM((CH,), jnp.int32),
        pltpu.VMEM((CH,), jnp.int32),
    ]

    @functools.partial(
        pl.kernel,
        out_type=jax.ShapeDtypeStruct((B, C, S, S), jnp.float32),
        mesh=mesh,
        compiler_params=pltpu.CompilerParams(
            use_tc_tiling_on_sc=False, needs_layout_passes=False),
        scratch_types=[
            *chunk_buf,
            *chunk_buf,
            pltpu.VMEM((CH + LANES,), jnp.int32),
            *[pltpu.VMEM((PIX_W,), jnp.float32) for _ in range(K)],
            *[pltpu.VMEM((PIX_W,), jnp.int32) for _ in range(K)],
            pltpu.VMEM((C, ROWS_W, S), jnp.float32),
            pltpu.VMEM((K * LANES,), jnp.int32),
            pltpu.VMEM((K * LANES,), jnp.int32),
            pltpu.VMEM((K * LANES, C), jnp.float32),
            pltpu.VMEM((K * LANES, C), jnp.float32),
            pltpu.SemaphoreType.DMA,
            pltpu.SemaphoreType.DMA,
            pltpu.SemaphoreType.DMA,
            pltpu.SemaphoreType.DMA,
        ],
    )
    def raster(x_hbm, y_hbm, z_hbm, ci_hbm, cj_hbm, ft_hbm, out_hbm,
               xbA, ybA, zbA, cibA, cjbA,
               xbB, ybB, zbB, cibB, cjbB,
               hid,
               zb0, zb1, zb2, zb3, zb4, zb5, zb6, zb7,
               pb0, pb1, pb2, pb3, pb4, pb5, pb6, pb7,
               acc, idxgA, idxgB, rowsA, rowsB,
               semA, semB, semRA, semRB):
        zbufs = (zb0, zb1, zb2, zb3, zb4, zb5, zb6, zb7)
        pbufs = (pb0, pb1, pb2, pb3, pb4, pb5, pb6, pb7)
        cid = lax.axis_index("c")
        sid = lax.axis_index("s")
        wid = sid * NC + cid
        lo = wid * ROWS_W
        iota = lax.broadcasted_iota(jnp.int32, (LANES,), 0)
        groups = []
        for g0 in range(2):
            off = iota + g0 * LANES
            oiv = off // 5 - 2
            ojv = lax.rem(off, 5) - 2
            gm = off < 25
            groups.append((oiv, ojv, gm))
        zero16f = jnp.zeros((LANES,), jnp.float32)
        inf16 = jnp.full((LANES,), jnp.inf, jnp.float32)
        zero16i = jnp.zeros((LANES,), jnp.int32)

        bufsA = (xbA, ybA, zbA, cibA, cjbA)
        bufsB = (xbB, ybB, zbB, cibB, cjbB)
        hbm5 = (x_hbm, y_hbm, z_hbm, ci_hbm, cj_hbm)

        def issue_chunk(b, t, bufs, sem):
            src = pl.ds(t * CH, CH)
            for h, d in zip(hbm5, bufs):
                pltpu.async_copy(h.at[b, src], d, sem)

        def wait_chunk(b, t, bufs, sem):
            src = pl.ds(t * CH, CH)
            for h, d in zip(hbm5, bufs):
                pltpu.make_async_copy(h.at[b, src], d, sem).wait()

        def batch_body(b, _):
            boff = b * N

            def init_body(i, _c):
                sl = pl.ds(i * LANES, LANES)
                for k in range(K):
                    zbufs[k][sl] = inf16
                    pbufs[k][sl] = zero16i
                return _c

            lax.fori_loop(0, PIX_W // LANES, init_body, 0)

            def make_hit_body(t, bufs):
                xb, yb, zb, cib, cjb = bufs

                def hit_body(h, _c):
                    hsp = jnp.full((LANES,), h, jnp.int32)
                    lidx = plsc.load_gather(hid, [hsp])
                    px = plsc.load_gather(xb, [lidx])
                    py = plsc.load_gather(yb, [lidx])
                    pz = plsc.load_gather(zb, [lidx])
                    pci = plsc.load_gather(cib, [lidx])
                    pcj = plsc.load_gather(cjb, [lidx])
                    pidv = lidx + t * CH
                    _hit_tail(px, py, pz, pci, pcj, pidv)
                    return _c

                return hit_body

            def _hit_tail(px, py, pz, pci, pcj, pidv):
                for (oiv, ojv, gm) in groups:
                    pi = pci + oiv
                    pj = pcj + ojv
                    m = (gm & (pi >= lo) & (pi < lo + ROWS_W)
                         & (pj >= 0) & (pj < S))
                    cxv = 1.0 - (pj.astype(jnp.float32) + 0.5) * (2.0 / S)
                    cyv = 1.0 - (pi.astype(jnp.float32) + 0.5) * (2.0 / S)
                    dx = px - cxv
                    dy = py - cyv
                    d2 = dx * dx + dy * dy
                    m = m & (d2 < R2)
                    cd = jnp.clip(d2 * (1.0 / R2), 0.001, 1.0)
                    yv = plsc.bitcast(
                        (plsc.bitcast(cd, jnp.int32) >> 1) + 0x1FBD1DF5,
                        jnp.float32)
                    for _ in range(3):
                        yv = 0.5 * (yv + cd / yv)
                    al = 1.0 - yv
                    aq = (al * 65535.0 + 0.5).astype(jnp.int32)
                    p = (pi - lo) * S + pj
                    p = jnp.where(m, p, 0)
                    zc = pz
                    pc = (pidv << 16) | aq
                    for k in range(K):
                        zk = plsc.load_gather(zbufs[k], [p], mask=m)
                        sw = m & (zc < zk)
                        plsc.store_scatter(zbufs[k], [p], zc, mask=sw)
                        pk_ = plsc.load_gather(pbufs[k], [p], mask=sw)
                        plsc.store_scatter(pbufs[k], [p], pc, mask=sw)
                        zc = jnp.where(sw, zk, zc)
                        pc = jnp.where(sw, pk_, pc)

            def process_chunk(t, bufs):
                xb, yb, zb, cib, cjb = bufs

                def scan_body(g, cnt):
                    sl = pl.ds(g * LANES, LANES)
                    civ = cib[sl]
                    cjv = cjb[sl]
                    zv = zb[sl]
                    m = ((civ >= lo - 2) & (civ <= lo + ROWS_W + 1)
                         & (zv > 0.0) & (cjv >= -2) & (cjv <= S + 1))
                    lidx = g * LANES + iota
                    plsc.store_compressed(hid.at[pl.ds(cnt, LANES)],
                                          lidx, mask=m)
                    return cnt + _scalar(
                        plsc.all_reduce_population_count(m))

                cnt = lax.fori_loop(0, CH // LANES, scan_body,
                                    jnp.int32(0))
                lax.fori_loop(0, cnt, make_hit_body(t, bufs), 0)

            def pair_body(t, _c):
                wait_chunk(b, 2 * t, bufsA, semA)
                issue_chunk(b, 2 * t + 1, bufsB, semB)
                process_chunk(2 * t, bufsA)
                wait_chunk(b, 2 * t + 1, bufsB, semB)

                @pl.when(t < N // CH // 2 - 1)
                def _():
                    issue_chunk(b, 2 * t + 2, bufsA, semA)

                process_chunk(2 * t + 1, bufsB)
                return _c

            issue_chunk(b, 0, bufsA, semA)
            lax.fori_loop(0, N // CH // 2, pair_body, 0)

            def build_idx(g, idxg):
                base = g * LANES
                for k in range(K):
                    pv = pbufs[k][pl.ds(base, LANES)]
                    idxg[pl.ds(k * LANES, LANES)] = (
                        lax.shift_right_logical(pv, 16) + boff)

            def compute_group(g, rows):
                base = g * LANES
                avs = [(pbufs[k][pl.ds(base, LANES)] & 0xFFFF
                        ).astype(jnp.float32) * (1.0 / 65535.0)
                       for k in range(K)]
                rg = base // S
                cb = lax.rem(base, S)
                for c in range(C):
                    accv = zero16f
                    csp = jnp.full((LANES,), c, jnp.int32)
                    for k in range(K):
                        fv = plsc.load_gather(
                            rows, [iota + k * LANES, csp])
                        accv = accv + avs[k] * fv
                    acc[c, rg, pl.ds(cb, LANES)] = accv

            def comp_pair(g, _c):
                build_idx(2 * g + 1, idxgB)
                pltpu.async_copy(ft_hbm.at[idxgB], rowsB, semRB)
                pltpu.make_async_copy(ft_hbm.at[idxgA], rowsA,
                                      semRA).wait()
                compute_group(2 * g, rowsA)

                @pl.when(g < PIX_W // LANES // 2 - 1)
                def _():
                    build_idx(2 * g + 2, idxgA)
                    pltpu.async_copy(ft_hbm.at[idxgA], rowsA, semRA)

                pltpu.make_async_copy(ft_hbm.at[idxgB], rowsB,
                                      semRB).wait()
                compute_group(2 * g + 1, rowsB)
                return _c

            build_idx(0, idxgA)
            pltpu.async_copy(ft_hbm.at[idxgA], rowsA, semRA)
            lax.fori_loop(0, PIX_W // LANES // 2, comp_pair, 0)
            pltpu.sync_copy(acc, out_hbm.at[b, :, pl.ds(lo, ROWS_W), :])
            return _

        lax.fori_loop(0, B, batch_body, 0)

    return raster


@jax.jit
def kernel(pts3D, src):
    B, N = pts3D.shape[0], pts3D.shape[1]
    x = -pts3D[..., 0]
    y = -pts3D[..., 1]
    z = pts3D[..., 2]
    jf = (1.0 - x) * (S / 2.0) - 0.5
    if_ = (1.0 - y) * (S / 2.0) - 0.5
    cj = jnp.round(jf).astype(jnp.int32)
    ci = jnp.round(if_).astype(jnp.int32)
    ft = jnp.transpose(src, (0, 2, 1)).reshape(B * N, C)
    return _make_raster(B, N)(x, y, z, ci, cj, ft)


# speedup vs baseline: 1.0653x; 1.0653x over previous
import functools

import jax
import jax.numpy as jnp
from jax import lax
from jax.experimental import pallas as pl
from jax.experimental.pallas import tpu as pltpu
from jax.experimental.pallas import tpu_sc as plsc

S = 256
K = 8
C = 16
LANES = 16
NC, NS = 2, 16
NW = NC * NS
ROWS_W = S // NW
PIX_W = ROWS_W * S
CH = 2048
R_NDC = 1.5 / S * 2.0
R2 = R_NDC * R_NDC


def _scalar(v):
    v = jnp.asarray(v)
    return v[0] if v.ndim == 1 else v


@functools.lru_cache(maxsize=None)
def _make_raster(B, N):
    mesh = plsc.VectorSubcoreMesh(
        core_axis_name="c", subcore_axis_name="s",
        num_cores=NC, num_subcores=NS)

    chunk_buf = [
        pltpu.VMEM((CH,), jnp.float32),
        pltpu.VMEM((CH,), jnp.float32),
        pltpu.VMEM((CH,), jnp.float32),
        pltpu.VMEM((CH,), jnp.int32),
        pltpu.VMEM((CH,), jnp.int32),
    ]

    @functools.partial(
        pl.kernel,
        out_type=jax.ShapeDtypeStruct((B, C, S, S), jnp.float32),
        mesh=mesh,
        compiler_params=pltpu.CompilerParams(
            use_tc_tiling_on_sc=False, needs_layout_passes=False),
        scratch_types=[
            *chunk_buf,
            *chunk_buf,
            pltpu.VMEM((CH + LANES,), jnp.int32),
            *[pltpu.VMEM((PIX_W,), jnp.float32) for _ in range(K)],
            *[pltpu.VMEM((PIX_W,), jnp.int32) for _ in range(K)],
            pltpu.VMEM((C, ROWS_W, S), jnp.float32),
            pltpu.VMEM((K * LANES,), jnp.int32),
            pltpu.VMEM((K * LANES,), jnp.int32),
            pltpu.VMEM((K * LANES, C), jnp.float32),
            pltpu.VMEM((K * LANES, C), jnp.float32),
            pltpu.SemaphoreType.DMA,
            pltpu.SemaphoreType.DMA,
            pltpu.SemaphoreType.DMA,
            pltpu.SemaphoreType.DMA,
        ],
    )
    def raster(x_hbm, y_hbm, z_hbm, ci_hbm, cj_hbm, ft_hbm, out_hbm,
               xbA, ybA, zbA, cibA, cjbA,
               xbB, ybB, zbB, cibB, cjbB,
               hid,
               zb0, zb1, zb2, zb3, zb4, zb5, zb6, zb7,
               pb0, pb1, pb2, pb3, pb4, pb5, pb6, pb7,
               acc, idxgA, idxgB, rowsA, rowsB,
               semA, semB, semRA, semRB):
        zbufs = (zb0, zb1, zb2, zb3, zb4, zb5, zb6, zb7)
        pbufs = (pb0, pb1, pb2, pb3, pb4, pb5, pb6, pb7)
        cid = lax.axis_index("c")
        sid = lax.axis_index("s")
        wid = sid * NC + cid
        lo = wid * ROWS_W
        iota = lax.broadcasted_iota(jnp.int32, (LANES,), 0)
        groups = []
        for g0 in range(2):
            off = iota + g0 * LANES
            oiv = off // 5 - 2
            ojv = lax.rem(off, 5) - 2
            gm = off < 25
            groups.append((oiv, ojv, gm))
        zero16f = jnp.zeros((LANES,), jnp.float32)
        inf16 = jnp.full((LANES,), jnp.inf, jnp.float32)
        zero16i = jnp.zeros((LANES,), jnp.int32)

        bufsA = (xbA, ybA, zbA, cibA, cjbA)
        bufsB = (xbB, ybB, zbB, cibB, cjbB)
        hbm5 = (x_hbm, y_hbm, z_hbm, ci_hbm, cj_hbm)

        def issue_chunk(b, t, bufs, sem):
            src = pl.ds(t * CH, CH)
            for h, d in zip(hbm5, bufs):
                pltpu.async_copy(h.at[b, src], d, sem)

        def wait_chunk(b, t, bufs, sem):
            src = pl.ds(t * CH, CH)
            for h, d in zip(hbm5, bufs):
                pltpu.make_async_copy(h.at[b, src], d, sem).wait()

        def batch_body(b, _):
            boff = b * N

            def init_body(i, _c):
                sl = pl.ds(i * LANES, LANES)
                for k in range(K):
                    zbufs[k][sl] = inf16
                    pbufs[k][sl] = zero16i
                return _c

            lax.fori_loop(0, PIX_W // LANES, init_body, 0)

            def make_hit_body(t, bufs):
                xb, yb, zb, cib, cjb = bufs

                def hit_body(h, _c):
                    hsp = jnp.full((LANES,), h, jnp.int32)
                    lidx = plsc.load_gather(hid, [hsp])
                    px = plsc.load_gather(xb, [lidx])
                    py = plsc.load_gather(yb, [lidx])
                    pz = plsc.load_gather(zb, [lidx])
                    pci = plsc.load_gather(cib, [lidx])
                    pcj = plsc.load_gather(cjb, [lidx])
                    pidv = lidx + t * CH
                    _hit_tail(px, py, pz, pci, pcj, pidv)
                    return _c

                return hit_body

            def _hit_tail(px, py, pz, pci, pcj, pidv):
                for (oiv, ojv, gm) in groups:
                    pi = pci + oiv
                    pj = pcj + ojv
                    m = (gm & (pi >= lo) & (pi < lo + ROWS_W)
                         & (pj >= 0) & (pj < S))
                    cxv = 1.0 - (pj.astype(jnp.float32) + 0.5) * (2.0 / S)
                    cyv = 1.0 - (pi.astype(jnp.float32) + 0.5) * (2.0 / S)
                    dx = px - cxv
                    dy = py - cyv
                    d2 = dx * dx + dy * dy
                    m = m & (d2 < R2)
                    cd = jnp.clip(d2 * (1.0 / R2), 0.001, 1.0)
                    yv = plsc.bitcast(
                        (plsc.bitcast(cd, jnp.int32) >> 1) + 0x1FBD1DF5,
                        jnp.float32)
                    for _ in range(2):
                        yv = 0.5 * (yv + cd / yv)
                    al = 1.0 - yv
                    aq = (al * 65535.0 + 0.5).astype(jnp.int32)
                    p = (pi - lo) * S + pj
                    p = jnp.where(m, p, 0)
                    zc = pz
                    pc = (pidv << 16) | aq
                    zks = [plsc.load_gather(zbufs[k], [p], mask=m)
                           for k in range(K)]
                    pks = [plsc.load_gather(pbufs[k], [p], mask=m)
                           for k in range(K)]
                    for k in range(K):
                        sw = m & (zc < zks[k])
                        plsc.store_scatter(zbufs[k], [p], zc, mask=sw)
                        plsc.store_scatter(pbufs[k], [p], pc, mask=sw)
                        zc = jnp.where(sw, zks[k], zc)
                        pc = jnp.where(sw, pks[k], pc)

            def process_chunk(t, bufs):
                xb, yb, zb, cib, cjb = bufs

                def scan_body(g, cnt):
                    sl = pl.ds(g * LANES, LANES)
                    civ = cib[sl]
                    cjv = cjb[sl]
                    zv = zb[sl]
                    m = ((civ >= lo - 2) & (civ <= lo + ROWS_W + 1)
                         & (zv > 0.0) & (cjv >= -2) & (cjv <= S + 1))
                    lidx = g * LANES + iota
                    plsc.store_compressed(hid.at[pl.ds(cnt, LANES)],
                                          lidx, mask=m)
                    return cnt + _scalar(
                        plsc.all_reduce_population_count(m))

                cnt = lax.fori_loop(0, CH // LANES, scan_body,
                                    jnp.int32(0))
                lax.fori_loop(0, cnt, make_hit_body(t, bufs), 0)

            def pair_body(t, _c):
                wait_chunk(b, 2 * t, bufsA, semA)
                issue_chunk(b, 2 * t + 1, bufsB, semB)
                process_chunk(2 * t, bufsA)
                wait_chunk(b, 2 * t + 1, bufsB, semB)

                @pl.when(t < N // CH // 2 - 1)
                def _():
                    issue_chunk(b, 2 * t + 2, bufsA, semA)

                process_chunk(2 * t + 1, bufsB)
                return _c

            issue_chunk(b, 0, bufsA, semA)
            lax.fori_loop(0, N // CH // 2, pair_body, 0)

            def build_idx(g, idxg):
                base = g * LANES
                for k in range(K):
                    pv = pbufs[k][pl.ds(base, LANES)]
                    idxg[pl.ds(k * LANES, LANES)] = (
                        lax.shift_right_logical(pv, 16) + boff)

            def compute_group(g, rows):
                base = g * LANES
                avs = [(pbufs[k][pl.ds(base, LANES)] & 0xFFFF
                        ).astype(jnp.float32) * (1.0 / 65535.0)
                       for k in range(K)]
                rg = base // S
                cb = lax.rem(base, S)
                for c in range(C):
                    accv = zero16f
                    csp = jnp.full((LANES,), c, jnp.int32)
                    for k in range(K):
                        fv = plsc.load_gather(
                            rows, [iota + k * LANES, csp])
                        accv = accv + avs[k] * fv
                    acc[c, rg, pl.ds(cb, LANES)] = accv

            def comp_pair(g, _c):
                build_idx(2 * g + 1, idxgB)
                pltpu.async_copy(ft_hbm.at[idxgB], rowsB, semRB)
                pltpu.make_async_copy(ft_hbm.at[idxgA], rowsA,
                                      semRA).wait()
                compute_group(2 * g, rowsA)

                @pl.when(g < PIX_W // LANES // 2 - 1)
                def _():
                    build_idx(2 * g + 2, idxgA)
                    pltpu.async_copy(ft_hbm.at[idxgA], rowsA, semRA)

                pltpu.make_async_copy(ft_hbm.at[idxgB], rowsB,
                                      semRB).wait()
                compute_group(2 * g + 1, rowsB)
                return _c

            build_idx(0, idxgA)
            pltpu.async_copy(ft_hbm.at[idxgA], rowsA, semRA)
            lax.fori_loop(0, PIX_W // LANES // 2, comp_pair, 0)
            pltpu.sync_copy(acc, out_hbm.at[b, :, pl.ds(lo, ROWS_W), :])
            return _

        lax.fori_loop(0, B, batch_body, 0)

    return raster


@jax.jit
def kernel(pts3D, src):
    B, N = pts3D.shape[0], pts3D.shape[1]
    x = -pts3D[..., 0]
    y = -pts3D[..., 1]
    z = pts3D[..., 2]
    jf = (1.0 - x) * (S / 2.0) - 0.5
    if_ = (1.0 - y) * (S / 2.0) - 0.5
    cj = jnp.round(jf).astype(jnp.int32)
    ci = jnp.round(if_).astype(jnp.int32)
    ft = jnp.transpose(src, (0, 2, 1)).reshape(B * N, C)
    return _make_raster(B, N)(x, y, z, ci, cj, ft)
